# baseline (speedup 1.0000x reference)
import functools

import jax
import jax.numpy as jnp
from jax import lax
from jax.experimental import pallas as pl
from jax.experimental.pallas import tpu as pltpu
from jax.experimental.pallas import tpu_sc as plsc

N_NODES = 10000
HIDDEN = 128
NODE_DIMS = 64
TYPE_DIMS = 16
LENGTH_DIMS = 16
LANE_DIMS = 32
TYPE_NUM = 20
LENGTH_NUM = 100
LANE_NUM = 10

NUM_TILES = 32
EDGE_CHUNK = 64
NBUF = 4
GROUP = 16
CORE0_NUM = 1
CORE_DEN = 1
ROW_BLOCK = 1000
N_PAD_NODES = 10240
ROWS_PER_SUBCORE = N_PAD_NODES // 16


def _pad_edges(n_edges):
    per_tile_chunks = -(-n_edges // (NUM_TILES * EDGE_CHUNK))
    per_tile_chunks = -(-per_tile_chunks // GROUP) * GROUP
    k0 = (2 * per_tile_chunks * CORE0_NUM // CORE_DEN) // GROUP * GROUP
    k1 = 2 * per_tile_chunks - k0
    return k0, k1, NUM_TILES * per_tile_chunks * EDGE_CHUNK


def _enc_body(tix, lix, nix, net, ttab, ltab, lanetab, w_ref, out):
    w = w_ref[...]
    lane_w = jnp.dot(lanetab[...], w[0:LANE_DIMS],
                     preferred_element_type=jnp.float32)
    type_w = jnp.dot(ttab[...], w[LANE_DIMS:LANE_DIMS + TYPE_DIMS],
                     preferred_element_type=jnp.float32)
    len_w = jnp.dot(ltab[...], w[LANE_DIMS + TYPE_DIMS:
                                 LANE_DIMS + TYPE_DIMS + LENGTH_DIMS],
                    preferred_element_type=jnp.float32)
    acc = jnp.dot(net[...], w[HIDDEN - NODE_DIMS:],
                  preferred_element_type=jnp.float32)
    t = tix[0, 0, :]
    le = lix[0, 0, :]
    la = nix[0, 0, :]
    oh_t = (t[:, None] == lax.broadcasted_iota(
        jnp.int32, (ROW_BLOCK, TYPE_NUM), 1)).astype(jnp.float32)
    oh_le = (le[:, None] == lax.broadcasted_iota(
        jnp.int32, (ROW_BLOCK, LENGTH_NUM), 1)).astype(jnp.float32)
    oh_la = (la[:, None] == lax.broadcasted_iota(
        jnp.int32, (ROW_BLOCK, LANE_NUM), 1)).astype(jnp.float32)
    acc = acc + jnp.dot(oh_la, lane_w, preferred_element_type=jnp.float32)
    acc = acc + jnp.dot(oh_t, type_w, preferred_element_type=jnp.float32)
    acc = acc + jnp.dot(oh_le, len_w, preferred_element_type=jnp.float32)
    out[...] = acc


def _encode_support(type_f, length_f, lane_f, node_emb_table,
                    type_tab, len_tab, lane_tab, w):
    n_blocks = N_NODES // ROW_BLOCK
    full = lambda shape: pl.BlockSpec(shape, lambda j: (0,) * len(shape))
    return pl.pallas_call(
        _enc_body,
        grid=(n_blocks,),
        in_specs=[
            pl.BlockSpec((1, 1, ROW_BLOCK), lambda j: (j, 0, 0)),
            pl.BlockSpec((1, 1, ROW_BLOCK), lambda j: (j, 0, 0)),
            pl.BlockSpec((1, 1, ROW_BLOCK), lambda j: (j, 0, 0)),
            pl.BlockSpec((ROW_BLOCK, NODE_DIMS), lambda j: (j, 0)),
            full((TYPE_NUM, TYPE_DIMS)),
            full((LENGTH_NUM, LENGTH_DIMS)),
            full((LANE_NUM, LANE_DIMS)),
            full((HIDDEN, HIDDEN)),
        ],
        out_specs=pl.BlockSpec((ROW_BLOCK, HIDDEN), lambda j: (j, 0)),
        out_shape=jax.ShapeDtypeStruct((N_NODES, HIDDEN), jnp.float32),
    )(type_f.reshape(n_blocks, 1, ROW_BLOCK),
      length_f.reshape(n_blocks, 1, ROW_BLOCK),
      lane_f.reshape(n_blocks, 1, ROW_BLOCK),
      node_emb_table, type_tab, len_tab, lane_tab, w)


def _sum_parts(agg, b):
    acc = agg[0]
    for p in range(1, agg.shape[0]):
        acc = acc + agg[p]
    return jnp.maximum(acc + b[...], 0.0)


def _mid_body(agg, b, w_ref, out):
    h = _sum_parts(agg[...], b)
    out[...] = jnp.dot(h, w_ref[...], preferred_element_type=jnp.float32)


def _mid_layer(parts, b2, w):
    n_blocks = N_NODES // ROW_BLOCK
    n_parts = parts.shape[0]
    return pl.pallas_call(
        _mid_body,
        grid=(n_blocks,),
        in_specs=[
            pl.BlockSpec((n_parts, ROW_BLOCK, HIDDEN), lambda j: (0, j, 0)),
            pl.BlockSpec((1, HIDDEN), lambda j: (0, 0)),
            pl.BlockSpec((HIDDEN, HIDDEN), lambda j: (0, 0)),
        ],
        out_specs=pl.BlockSpec((ROW_BLOCK, HIDDEN), lambda j: (j, 0)),
        out_shape=jax.ShapeDtypeStruct((N_NODES, HIDDEN), jnp.float32),
    )(parts, b2, w)


def _fin_body(agg, b, out):
    out[...] = _sum_parts(agg[...], b)


def _final_layer(parts, b2):
    n_blocks = N_NODES // ROW_BLOCK
    n_parts = parts.shape[0]
    return pl.pallas_call(
        _fin_body,
        grid=(n_blocks,),
        in_specs=[
            pl.BlockSpec((n_parts, ROW_BLOCK, HIDDEN), lambda j: (0, j, 0)),
            pl.BlockSpec((1, HIDDEN), lambda j: (0, 0)),
        ],
        out_specs=pl.BlockSpec((ROW_BLOCK, HIDDEN), lambda j: (j, 0)),
        out_shape=jax.ShapeDtypeStruct((N_NODES, HIDDEN), jnp.float32),
    )(parts, b2)


def _make_spmm(k0, k1):
    mesh = plsc.VectorSubcoreMesh(core_axis_name="c", subcore_axis_name="s")
    n_parts = 2 if k1 else 1

    @functools.partial(
        pl.kernel,
        mesh=mesh,
        compiler_params=pltpu.CompilerParams(use_tc_tiling_on_sc=False),
        out_type=jax.ShapeDtypeStruct((n_parts, N_PAD_NODES, HIDDEN),
                                      jnp.float32),
        scratch_types=[
            pltpu.VMEM((GROUP, EDGE_CHUNK), jnp.int32),
            pltpu.VMEM((GROUP, EDGE_CHUNK), jnp.int32),
            pltpu.VMEM((GROUP, EDGE_CHUNK), jnp.float32),
        ] + [pltpu.VMEM((EDGE_CHUNK, HIDDEN), jnp.float32)
             for _ in range(NBUF)] + [
            pltpu.VMEM_SHARED((N_PAD_NODES, HIDDEN), jnp.float32),
        ] + [pltpu.SemaphoreType.DMA for _ in range(NBUF)],
    )
    def spmm(support_hbm, src_hbm, dst_hbm, w_hbm, out_hbm,
             src_v, dst_v, w_v, *rest):
        bufs = rest[:NBUF]
        agg = rest[NBUF]
        sems = rest[NBUF + 1:]
        c = lax.axis_index("c")
        s = lax.axis_index("s")

        rbase = s * ROWS_PER_SUBCORE

        def init_agg():
            zeros16 = jnp.zeros((16,), jnp.float32)

            def zero_body(i, carry):
                for cc in range(HIDDEN // 16):
                    bufs[0][i, pl.ds(cc * 16, 16)] = zeros16
                return carry

            lax.fori_loop(0, EDGE_CHUNK, zero_body, 0)
            for zz in range(ROWS_PER_SUBCORE // EDGE_CHUNK):
                pltpu.sync_copy(
                    bufs[0],
                    agg.at[pl.ds(rbase + zz * EDGE_CHUNK, EDGE_CHUNK)])

        if k1:
            init_agg()
        else:
            pl.when(c == 0)(init_agg)

        plsc.subcore_barrier()

        ebase = jnp.where(c == 0, s * k0, 16 * k0 + s * k1)
        n_groups = jnp.where(c == 0, k0 // GROUP, k1 // GROUP)

        def process(j, b):
            rbuf = bufs[b]
            pltpu.make_async_copy(support_hbm.at[src_v.at[j]], rbuf,
                                  sems[b]).wait()

            def mul_body(g, carry):
                wvec = w_v[j, pl.ds(g * 16, 16)]
                for k in range(16):
                    wi = wvec[k]
                    i = g * 16 + k
                    for cc in range(HIDDEN // 16):
                        sl = pl.ds(cc * 16, 16)
                        rbuf[i, sl] = rbuf[i, sl] * wi
                return carry

            lax.fori_loop(0, EDGE_CHUNK // 16, mul_body, 0)
            pltpu.sync_copy(rbuf, agg.at[dst_v.at[j]], add=True)

            nb = (b + NBUF - 1) % NBUF
            @pl.when(j + NBUF - 1 < GROUP)
            def _():
                pltpu.async_copy(support_hbm.at[src_v.at[j + NBUF - 1]],
                                 bufs[nb], sems[nb])

        def group_body(g, carry):
            gb = ebase + g * GROUP
            pltpu.sync_copy(src_hbm.at[pl.ds(gb, GROUP)], src_v)
            pltpu.sync_copy(dst_hbm.at[pl.ds(gb, GROUP)], dst_v)
            pltpu.sync_copy(w_hbm.at[pl.ds(gb, GROUP)], w_v)
            for j in range(NBUF - 1):
                pltpu.async_copy(support_hbm.at[src_v.at[j]], bufs[j],
                                 sems[j])

            def inner(k, c2):
                for b in range(NBUF):
                    process(NBUF * k + b, b)
                return c2

            lax.fori_loop(0, GROUP // NBUF, inner, 0)
            return carry

        lax.fori_loop(0, n_groups, group_body, 0)

        plsc.subcore_barrier()

        def dump_agg():
            pltpu.sync_copy(agg.at[pl.ds(rbase, ROWS_PER_SUBCORE)],
                            out_hbm.at[c, pl.ds(rbase, ROWS_PER_SUBCORE)])

        if k1:
            dump_agg()
        else:
            pl.when(c == 0)(dump_agg)

    return spmm


def kernel(node_feature, type_feature, length_feature, lane_feature,
           edge_index, edge_weight,
           node_emb_table, type_emb_table, length_emb_table, lane_emb_table,
           W, b):
    del node_feature
    n_edges = edge_index.shape[1]
    k0, k1, n_pad_edges = _pad_edges(n_edges)
    pad = n_pad_edges - n_edges

    src = jnp.concatenate(
        [edge_index[0].astype(jnp.int32), jnp.zeros((pad,), jnp.int32)])
    dst = jnp.concatenate(
        [edge_index[1].astype(jnp.int32), jnp.zeros((pad,), jnp.int32)])
    w_e = jnp.concatenate(
        [edge_weight.astype(jnp.float32), jnp.zeros((pad,), jnp.float32)])
    rows = 16 * (k0 + k1)
    src = src.reshape(rows, EDGE_CHUNK)
    dst = dst.reshape(rows, EDGE_CHUNK)
    w_e = w_e.reshape(rows, EDGE_CHUNK)
    b2 = b.reshape(1, HIDDEN).astype(jnp.float32)

    spmm = _make_spmm(k0, k1)

    support = _encode_support(
        type_feature.astype(jnp.int32), length_feature.astype(jnp.int32),
        lane_feature.astype(jnp.int32), node_emb_table.astype(jnp.float32),
        type_emb_table.astype(jnp.float32),
        length_emb_table.astype(jnp.float32),
        lane_emb_table.astype(jnp.float32), W.astype(jnp.float32))
    parts = spmm(support, src, dst, w_e)
    support2 = _mid_layer(parts, b2, W.astype(jnp.float32))
    parts2 = spmm(support2, src, dst, w_e)
    return _final_layer(parts2, b2)


# speedup vs baseline: 1.4358x; 1.4358x over previous
import functools

import jax
import jax.numpy as jnp
from jax import lax
from jax.experimental import pallas as pl
from jax.experimental.pallas import tpu as pltpu
from jax.experimental.pallas import tpu_sc as plsc

N_NODES = 10000
HIDDEN = 128
NODE_DIMS = 64
TYPE_DIMS = 16
LENGTH_DIMS = 16
LANE_DIMS = 32
TYPE_NUM = 20
LENGTH_NUM = 100
LANE_NUM = 10

NUM_TILES = 32
EDGE_CHUNK = 64
NBUF = 4
GROUP = 16
CORE0_NUM = 3
CORE_DEN = 4
ROW_BLOCK = 1000
N_PAD_NODES = 10240
ROWS_PER_SUBCORE = N_PAD_NODES // 16


def _pad_edges(n_edges):
    per_tile_chunks = -(-n_edges // (NUM_TILES * EDGE_CHUNK))
    per_tile_chunks = -(-per_tile_chunks // GROUP) * GROUP
    k0 = (2 * per_tile_chunks * CORE0_NUM // CORE_DEN) // GROUP * GROUP
    k1 = 2 * per_tile_chunks - k0
    return k0, k1, NUM_TILES * per_tile_chunks * EDGE_CHUNK


def _enc_body(tix, lix, nix, net, ttab, ltab, lanetab, w_ref, out):
    w = w_ref[...]
    lane_w = jnp.dot(lanetab[...], w[0:LANE_DIMS],
                     preferred_element_type=jnp.float32)
    type_w = jnp.dot(ttab[...], w[LANE_DIMS:LANE_DIMS + TYPE_DIMS],
                     preferred_element_type=jnp.float32)
    len_w = jnp.dot(ltab[...], w[LANE_DIMS + TYPE_DIMS:
                                 LANE_DIMS + TYPE_DIMS + LENGTH_DIMS],
                    preferred_element_type=jnp.float32)
    acc = jnp.dot(net[...], w[HIDDEN - NODE_DIMS:],
                  preferred_element_type=jnp.float32)
    t = tix[0, 0, :]
    le = lix[0, 0, :]
    la = nix[0, 0, :]
    oh_t = (t[:, None] == lax.broadcasted_iota(
        jnp.int32, (ROW_BLOCK, TYPE_NUM), 1)).astype(jnp.float32)
    oh_le = (le[:, None] == lax.broadcasted_iota(
        jnp.int32, (ROW_BLOCK, LENGTH_NUM), 1)).astype(jnp.float32)
    oh_la = (la[:, None] == lax.broadcasted_iota(
        jnp.int32, (ROW_BLOCK, LANE_NUM), 1)).astype(jnp.float32)
    acc = acc + jnp.dot(oh_la, lane_w, preferred_element_type=jnp.float32)
    acc = acc + jnp.dot(oh_t, type_w, preferred_element_type=jnp.float32)
    acc = acc + jnp.dot(oh_le, len_w, preferred_element_type=jnp.float32)
    out[...] = acc


def _encode_support(type_f, length_f, lane_f, node_emb_table,
                    type_tab, len_tab, lane_tab, w):
    n_blocks = N_NODES // ROW_BLOCK
    full = lambda shape: pl.BlockSpec(shape, lambda j: (0,) * len(shape))
    return pl.pallas_call(
        _enc_body,
        grid=(n_blocks,),
        in_specs=[
            pl.BlockSpec((1, 1, ROW_BLOCK), lambda j: (j, 0, 0)),
            pl.BlockSpec((1, 1, ROW_BLOCK), lambda j: (j, 0, 0)),
            pl.BlockSpec((1, 1, ROW_BLOCK), lambda j: (j, 0, 0)),
            pl.BlockSpec((ROW_BLOCK, NODE_DIMS), lambda j: (j, 0)),
            full((TYPE_NUM, TYPE_DIMS)),
            full((LENGTH_NUM, LENGTH_DIMS)),
            full((LANE_NUM, LANE_DIMS)),
            full((HIDDEN, HIDDEN)),
        ],
        out_specs=pl.BlockSpec((ROW_BLOCK, HIDDEN), lambda j: (j, 0)),
        out_shape=jax.ShapeDtypeStruct((N_NODES, HIDDEN), jnp.float32),
    )(type_f.reshape(n_blocks, 1, ROW_BLOCK),
      length_f.reshape(n_blocks, 1, ROW_BLOCK),
      lane_f.reshape(n_blocks, 1, ROW_BLOCK),
      node_emb_table, type_tab, len_tab, lane_tab, w)


def _sum_parts(agg, b):
    acc = agg[0]
    for p in range(1, agg.shape[0]):
        acc = acc + agg[p]
    return jnp.maximum(acc + b[...], 0.0)


def _mid_body(agg, b, w_ref, out):
    h = _sum_parts(agg[...], b)
    out[...] = jnp.dot(h, w_ref[...], preferred_element_type=jnp.float32)


def _mid_layer(parts, b2, w):
    n_blocks = N_NODES // ROW_BLOCK
    n_parts = parts.shape[0]
    return pl.pallas_call(
        _mid_body,
        grid=(n_blocks,),
        in_specs=[
            pl.BlockSpec((n_parts, ROW_BLOCK, HIDDEN), lambda j: (0, j, 0)),
            pl.BlockSpec((1, HIDDEN), lambda j: (0, 0)),
            pl.BlockSpec((HIDDEN, HIDDEN), lambda j: (0, 0)),
        ],
        out_specs=pl.BlockSpec((ROW_BLOCK, HIDDEN), lambda j: (j, 0)),
        out_shape=jax.ShapeDtypeStruct((N_NODES, HIDDEN), jnp.float32),
    )(parts, b2, w)


def _fin_body(agg, b, out):
    out[...] = _sum_parts(agg[...], b)


def _final_layer(parts, b2):
    n_blocks = N_NODES // ROW_BLOCK
    n_parts = parts.shape[0]
    return pl.pallas_call(
        _fin_body,
        grid=(n_blocks,),
        in_specs=[
            pl.BlockSpec((n_parts, ROW_BLOCK, HIDDEN), lambda j: (0, j, 0)),
            pl.BlockSpec((1, HIDDEN), lambda j: (0, 0)),
        ],
        out_specs=pl.BlockSpec((ROW_BLOCK, HIDDEN), lambda j: (j, 0)),
        out_shape=jax.ShapeDtypeStruct((N_NODES, HIDDEN), jnp.float32),
    )(parts, b2)


def _make_spmm(k0, k1):
    mesh = plsc.VectorSubcoreMesh(core_axis_name="c", subcore_axis_name="s")
    n_parts = 2 if k1 else 1

    @functools.partial(
        pl.kernel,
        mesh=mesh,
        compiler_params=pltpu.CompilerParams(use_tc_tiling_on_sc=False),
        out_type=jax.ShapeDtypeStruct((n_parts, N_PAD_NODES, HIDDEN),
                                      jnp.float32),
        scratch_types=[
            pltpu.VMEM((GROUP, EDGE_CHUNK), jnp.int32),
            pltpu.VMEM((GROUP, EDGE_CHUNK), jnp.int32),
            pltpu.VMEM((GROUP, EDGE_CHUNK), jnp.float32),
        ] + [pltpu.VMEM((EDGE_CHUNK, HIDDEN), jnp.float32)
             for _ in range(NBUF)] + [
            pltpu.VMEM_SHARED((N_PAD_NODES, HIDDEN), jnp.float32),
        ] + [pltpu.SemaphoreType.DMA for _ in range(NBUF)],
    )
    def spmm(support_hbm, src_hbm, dst_hbm, w_hbm, out_hbm,
             src_v, dst_v, w_v, *rest):
        bufs = rest[:NBUF]
        agg = rest[NBUF]
        sems = rest[NBUF + 1:]
        c = lax.axis_index("c")
        s = lax.axis_index("s")

        rbase = s * ROWS_PER_SUBCORE

        def init_agg():
            zeros16 = jnp.zeros((16,), jnp.float32)

            def zero_body(i, carry):
                for cc in range(HIDDEN // 16):
                    bufs[0][i, pl.ds(cc * 16, 16)] = zeros16
                return carry

            lax.fori_loop(0, EDGE_CHUNK, zero_body, 0)
            for zz in range(ROWS_PER_SUBCORE // EDGE_CHUNK):
                pltpu.sync_copy(
                    bufs[0],
                    agg.at[pl.ds(rbase + zz * EDGE_CHUNK, EDGE_CHUNK)])

        with jax.named_scope("agg_init"):
            if k1:
                init_agg()
            else:
                pl.when(c == 0)(init_agg)

        plsc.subcore_barrier()

        ebase = jnp.where(c == 0, s * k0, 16 * k0 + s * k1)
        n_groups = jnp.where(c == 0, k0 // GROUP, k1 // GROUP)

        def process(j, b):
            rbuf = bufs[b]
            pltpu.make_async_copy(support_hbm.at[src_v.at[j]], rbuf,
                                  sems[b]).wait()

            def mul_body(g, carry):
                wvec = w_v[j, pl.ds(g * 16, 16)]
                for k in range(16):
                    wi = wvec[k]
                    i = g * 16 + k
                    for cc in range(HIDDEN // 16):
                        sl = pl.ds(cc * 16, 16)
                        rbuf[i, sl] = rbuf[i, sl] * wi
                return carry

            lax.fori_loop(0, EDGE_CHUNK // 16, mul_body, 0)
            pltpu.sync_copy(rbuf, agg.at[dst_v.at[j]], add=True)

            nb = (b + NBUF - 1) % NBUF
            @pl.when(j + NBUF - 1 < GROUP)
            def _():
                pltpu.async_copy(support_hbm.at[src_v.at[j + NBUF - 1]],
                                 bufs[nb], sems[nb])

        def group_body(g, carry):
            gb = ebase + g * GROUP
            pltpu.sync_copy(src_hbm.at[pl.ds(gb, GROUP)], src_v)
            pltpu.sync_copy(dst_hbm.at[pl.ds(gb, GROUP)], dst_v)
            pltpu.sync_copy(w_hbm.at[pl.ds(gb, GROUP)], w_v)
            for j in range(NBUF - 1):
                pltpu.async_copy(support_hbm.at[src_v.at[j]], bufs[j],
                                 sems[j])

            def inner(k, c2):
                for b in range(NBUF):
                    process(NBUF * k + b, b)
                return c2

            lax.fori_loop(0, GROUP // NBUF, inner, 0)
            return carry

        with jax.named_scope("edge_loop"):
            lax.fori_loop(0, n_groups, group_body, 0)

        plsc.subcore_barrier()

        def dump_agg():
            pltpu.sync_copy(agg.at[pl.ds(rbase, ROWS_PER_SUBCORE)],
                            out_hbm.at[c, pl.ds(rbase, ROWS_PER_SUBCORE)])

        with jax.named_scope("agg_dump"):
            if k1:
                dump_agg()
            else:
                pl.when(c == 0)(dump_agg)

    return spmm


def kernel(node_feature, type_feature, length_feature, lane_feature,
           edge_index, edge_weight,
           node_emb_table, type_emb_table, length_emb_table, lane_emb_table,
           W, b):
    del node_feature
    n_edges = edge_index.shape[1]
    k0, k1, n_pad_edges = _pad_edges(n_edges)
    pad = n_pad_edges - n_edges

    src = jnp.concatenate(
        [edge_index[0].astype(jnp.int32), jnp.zeros((pad,), jnp.int32)])
    dst = jnp.concatenate(
        [edge_index[1].astype(jnp.int32), jnp.zeros((pad,), jnp.int32)])
    w_e = jnp.concatenate(
        [edge_weight.astype(jnp.float32), jnp.zeros((pad,), jnp.float32)])
    rows = 16 * (k0 + k1)
    src = src.reshape(rows, EDGE_CHUNK)
    dst = dst.reshape(rows, EDGE_CHUNK)
    w_e = w_e.reshape(rows, EDGE_CHUNK)
    b2 = b.reshape(1, HIDDEN).astype(jnp.float32)

    spmm = _make_spmm(k0, k1)

    support = _encode_support(
        type_feature.astype(jnp.int32), length_feature.astype(jnp.int32),
        lane_feature.astype(jnp.int32), node_emb_table.astype(jnp.float32),
        type_emb_table.astype(jnp.float32),
        length_emb_table.astype(jnp.float32),
        lane_emb_table.astype(jnp.float32), W.astype(jnp.float32))
    parts = spmm(support, src, dst, w_e)
    support2 = _mid_layer(parts, b2, W.astype(jnp.float32))
    parts2 = spmm(support2, src, dst, w_e)
    return _final_layer(parts2, b2)


# speedup vs baseline: 3.2973x; 2.2965x over previous
import functools

import jax
import jax.numpy as jnp
from jax import lax
from jax.experimental import pallas as pl
from jax.experimental.pallas import tpu as pltpu
from jax.experimental.pallas import tpu_sc as plsc

N_NODES = 10000
HIDDEN = 128
NODE_DIMS = 64
TYPE_DIMS = 16
LENGTH_DIMS = 16
LANE_DIMS = 32
TYPE_NUM = 20
LENGTH_NUM = 100
LANE_NUM = 10

NUM_TILES = 32
EDGE_CHUNK = 64
NBUF = 4
GROUP = 16
CORE0_NUM = 1
CORE_DEN = 2
ROW_BLOCK = 1000
N_PAD_NODES = 10240
ROWS_PER_SUBCORE = N_PAD_NODES // 16


def _pad_edges(n_edges):
    per_tile_chunks = -(-n_edges // (NUM_TILES * EDGE_CHUNK))
    per_tile_chunks = -(-per_tile_chunks // GROUP) * GROUP
    k0 = (2 * per_tile_chunks * CORE0_NUM // CORE_DEN) // GROUP * GROUP
    k1 = 2 * per_tile_chunks - k0
    return k0, k1, NUM_TILES * per_tile_chunks * EDGE_CHUNK


def _enc_body(tix, lix, nix, net, ttab, ltab, lanetab, w_ref, out):
    w = w_ref[...]
    lane_w = jnp.dot(lanetab[...], w[0:LANE_DIMS],
                     preferred_element_type=jnp.float32)
    type_w = jnp.dot(ttab[...], w[LANE_DIMS:LANE_DIMS + TYPE_DIMS],
                     preferred_element_type=jnp.float32)
    len_w = jnp.dot(ltab[...], w[LANE_DIMS + TYPE_DIMS:
                                 LANE_DIMS + TYPE_DIMS + LENGTH_DIMS],
                    preferred_element_type=jnp.float32)
    acc = jnp.dot(net[...], w[HIDDEN - NODE_DIMS:],
                  preferred_element_type=jnp.float32)
    t = tix[0, 0, :]
    le = lix[0, 0, :]
    la = nix[0, 0, :]
    oh_t = (t[:, None] == lax.broadcasted_iota(
        jnp.int32, (ROW_BLOCK, TYPE_NUM), 1)).astype(jnp.float32)
    oh_le = (le[:, None] == lax.broadcasted_iota(
        jnp.int32, (ROW_BLOCK, LENGTH_NUM), 1)).astype(jnp.float32)
    oh_la = (la[:, None] == lax.broadcasted_iota(
        jnp.int32, (ROW_BLOCK, LANE_NUM), 1)).astype(jnp.float32)
    acc = acc + jnp.dot(oh_la, lane_w, preferred_element_type=jnp.float32)
    acc = acc + jnp.dot(oh_t, type_w, preferred_element_type=jnp.float32)
    acc = acc + jnp.dot(oh_le, len_w, preferred_element_type=jnp.float32)
    out[...] = acc


def _encode_support(type_f, length_f, lane_f, node_emb_table,
                    type_tab, len_tab, lane_tab, w):
    n_blocks = N_NODES // ROW_BLOCK
    full = lambda shape: pl.BlockSpec(shape, lambda j: (0,) * len(shape))
    return pl.pallas_call(
        _enc_body,
        grid=(n_blocks,),
        in_specs=[
            pl.BlockSpec((1, 1, ROW_BLOCK), lambda j: (j, 0, 0)),
            pl.BlockSpec((1, 1, ROW_BLOCK), lambda j: (j, 0, 0)),
            pl.BlockSpec((1, 1, ROW_BLOCK), lambda j: (j, 0, 0)),
            pl.BlockSpec((ROW_BLOCK, NODE_DIMS), lambda j: (j, 0)),
            full((TYPE_NUM, TYPE_DIMS)),
            full((LENGTH_NUM, LENGTH_DIMS)),
            full((LANE_NUM, LANE_DIMS)),
            full((HIDDEN, HIDDEN)),
        ],
        out_specs=pl.BlockSpec((ROW_BLOCK, HIDDEN), lambda j: (j, 0)),
        out_shape=jax.ShapeDtypeStruct((N_NODES, HIDDEN), jnp.float32),
    )(type_f.reshape(n_blocks, 1, ROW_BLOCK),
      length_f.reshape(n_blocks, 1, ROW_BLOCK),
      lane_f.reshape(n_blocks, 1, ROW_BLOCK),
      node_emb_table, type_tab, len_tab, lane_tab, w)


def _sum_parts(agg, b):
    acc = agg[0]
    for p in range(1, agg.shape[0]):
        acc = acc + agg[p]
    return jnp.maximum(acc + b[...], 0.0)


def _mid_body(agg, b, w_ref, out):
    h = _sum_parts(agg[...], b)
    out[...] = jnp.dot(h, w_ref[...], preferred_element_type=jnp.float32)


def _mid_layer(parts, b2, w):
    n_blocks = N_NODES // ROW_BLOCK
    n_parts = parts.shape[0]
    return pl.pallas_call(
        _mid_body,
        grid=(n_blocks,),
        in_specs=[
            pl.BlockSpec((n_parts, ROW_BLOCK, HIDDEN), lambda j: (0, j, 0)),
            pl.BlockSpec((1, HIDDEN), lambda j: (0, 0)),
            pl.BlockSpec((HIDDEN, HIDDEN), lambda j: (0, 0)),
        ],
        out_specs=pl.BlockSpec((ROW_BLOCK, HIDDEN), lambda j: (j, 0)),
        out_shape=jax.ShapeDtypeStruct((N_NODES, HIDDEN), jnp.float32),
    )(parts, b2, w)


def _fin_body(agg, b, out):
    out[...] = _sum_parts(agg[...], b)


def _final_layer(parts, b2):
    n_blocks = N_NODES // ROW_BLOCK
    n_parts = parts.shape[0]
    return pl.pallas_call(
        _fin_body,
        grid=(n_blocks,),
        in_specs=[
            pl.BlockSpec((n_parts, ROW_BLOCK, HIDDEN), lambda j: (0, j, 0)),
            pl.BlockSpec((1, HIDDEN), lambda j: (0, 0)),
        ],
        out_specs=pl.BlockSpec((ROW_BLOCK, HIDDEN), lambda j: (j, 0)),
        out_shape=jax.ShapeDtypeStruct((N_NODES, HIDDEN), jnp.float32),
    )(parts, b2)


def _make_spmm(k0, k1):
    mesh = plsc.VectorSubcoreMesh(core_axis_name="c", subcore_axis_name="s")
    n_parts = 2 if k1 else 1

    @functools.partial(
        pl.kernel,
        mesh=mesh,
        compiler_params=pltpu.CompilerParams(use_tc_tiling_on_sc=False),
        out_type=jax.ShapeDtypeStruct((n_parts, N_PAD_NODES, HIDDEN),
                                      jnp.float32),
        scratch_types=[
            pltpu.VMEM((GROUP, EDGE_CHUNK), jnp.int32),
            pltpu.VMEM((GROUP, EDGE_CHUNK), jnp.int32),
            pltpu.VMEM((GROUP, EDGE_CHUNK), jnp.float32),
        ] + [pltpu.VMEM((EDGE_CHUNK, HIDDEN), jnp.float32)
             for _ in range(NBUF)] + [
            pltpu.VMEM_SHARED((N_PAD_NODES, HIDDEN), jnp.float32),
        ] + [pltpu.SemaphoreType.DMA for _ in range(NBUF)],
    )
    def spmm(support_hbm, src_hbm, dst_hbm, w_hbm, out_hbm,
             src_v, dst_v, w_v, *rest):
        bufs = rest[:NBUF]
        agg = rest[NBUF]
        sems = rest[NBUF + 1:]
        c = lax.axis_index("c")
        s = lax.axis_index("s")

        rbase = s * ROWS_PER_SUBCORE

        def init_agg():
            zeros16 = jnp.zeros((16,), jnp.float32)

            def zero_body(i, carry):
                for cc in range(HIDDEN // 16):
                    bufs[0][i, pl.ds(cc * 16, 16)] = zeros16
                return carry

            lax.fori_loop(0, EDGE_CHUNK, zero_body, 0)
            for zz in range(ROWS_PER_SUBCORE // EDGE_CHUNK):
                pltpu.sync_copy(
                    bufs[0],
                    agg.at[pl.ds(rbase + zz * EDGE_CHUNK, EDGE_CHUNK)])

        with jax.named_scope("agg_init"):
            if k1:
                init_agg()
            else:
                pl.when(c == 0)(init_agg)

        plsc.subcore_barrier()

        ebase = jnp.where(c == 0, s * k0, 16 * k0 + s * k1)
        n_groups = jnp.where(c == 0, k0 // GROUP, k1 // GROUP)

        def process(j, b):
            rbuf = bufs[b]
            pltpu.make_async_copy(support_hbm.at[src_v.at[j]], rbuf,
                                  sems[b]).wait()

            def mul_body(g, carry):
                wvec = w_v[j, pl.ds(g * 16, 16)]
                for k in range(16):
                    wi = wvec[k]
                    i = g * 16 + k
                    for cc in range(HIDDEN // 16):
                        sl = pl.ds(cc * 16, 16)
                        rbuf[i, sl] = rbuf[i, sl] * wi
                return carry

            lax.fori_loop(0, EDGE_CHUNK // 16, mul_body, 0)
            pltpu.sync_copy(rbuf, agg.at[dst_v.at[j]], add=True)

            nb = (b + NBUF - 1) % NBUF
            @pl.when(j + NBUF - 1 < GROUP)
            def _():
                pltpu.async_copy(support_hbm.at[src_v.at[j + NBUF - 1]],
                                 bufs[nb], sems[nb])

        def group_body(g, carry):
            gb = ebase + g * GROUP
            pltpu.sync_copy(src_hbm.at[pl.ds(gb, GROUP)], src_v)
            pltpu.sync_copy(dst_hbm.at[pl.ds(gb, GROUP)], dst_v)
            pltpu.sync_copy(w_hbm.at[pl.ds(gb, GROUP)], w_v)
            for j in range(NBUF - 1):
                pltpu.async_copy(support_hbm.at[src_v.at[j]], bufs[j],
                                 sems[j])

            def inner(k, c2):
                for b in range(NBUF):
                    process(NBUF * k + b, b)
                return c2

            lax.fori_loop(0, GROUP // NBUF, inner, 0)
            return carry

        with jax.named_scope("edge_loop"):
            lax.fori_loop(0, n_groups, group_body, 0)

        plsc.subcore_barrier()

        def dump_agg():
            pltpu.sync_copy(agg.at[pl.ds(rbase, ROWS_PER_SUBCORE)],
                            out_hbm.at[c, pl.ds(rbase, ROWS_PER_SUBCORE)])

        with jax.named_scope("agg_dump"):
            if k1:
                dump_agg()
            else:
                pl.when(c == 0)(dump_agg)

    return spmm


def kernel(node_feature, type_feature, length_feature, lane_feature,
           edge_index, edge_weight,
           node_emb_table, type_emb_table, length_emb_table, lane_emb_table,
           W, b):
    del node_feature
    n_edges = edge_index.shape[1]
    k0, k1, n_pad_edges = _pad_edges(n_edges)
    pad = n_pad_edges - n_edges

    pad_iota = jnp.arange(pad, dtype=jnp.int32)
    src = jnp.concatenate(
        [edge_index[0].astype(jnp.int32), pad_iota % N_NODES])
    dst = jnp.concatenate(
        [edge_index[1].astype(jnp.int32),
         N_NODES + pad_iota % (N_PAD_NODES - N_NODES)])
    w_e = jnp.concatenate(
        [edge_weight.astype(jnp.float32), jnp.zeros((pad,), jnp.float32)])
    rows = 16 * (k0 + k1)
    src = src.reshape(rows, EDGE_CHUNK)
    dst = dst.reshape(rows, EDGE_CHUNK)
    w_e = w_e.reshape(rows, EDGE_CHUNK)
    b2 = b.reshape(1, HIDDEN).astype(jnp.float32)

    spmm = _make_spmm(k0, k1)

    support = _encode_support(
        type_feature.astype(jnp.int32), length_feature.astype(jnp.int32),
        lane_feature.astype(jnp.int32), node_emb_table.astype(jnp.float32),
        type_emb_table.astype(jnp.float32),
        length_emb_table.astype(jnp.float32),
        lane_emb_table.astype(jnp.float32), W.astype(jnp.float32))
    parts = spmm(support, src, dst, w_e)
    support2 = _mid_layer(parts, b2, W.astype(jnp.float32))
    parts2 = spmm(support2, src, dst, w_e)
    return _final_layer(parts2, b2)
